# Initial kernel scaffold; baseline (speedup 1.0000x reference)
import jax
import jax.numpy as jnp
from jax.experimental import pallas as pl


def kernel(x, edge_index_l1, edge_index_l2, W1, a_src1, a_dst1, W2, a_src2, a_dst2):
    raise NotImplementedError("write your pallas kernel here")


# speedup vs baseline: 116.7381x; 116.7381x over previous
import functools

import jax
import jax.numpy as jnp
from jax import lax
from jax.experimental import pallas as pl
from jax.experimental.pallas import tpu as pltpu
from jax.experimental.pallas import tpu_sc as plsc

N = 10000
D = 128
DH = D // 2
E = 320000
NSTEP = 9
INV9 = 1.0 / 9.0

NCORES = 2
NSUB = 16
EPT = E // NSUB
RPT = 624
CHUNK = 80
NCHUNK = EPT // CHUNK
VPC = CHUNK // 16


def _prologue_body(x_ref, w_ref, a_ref, ha_ref, hb_ref, s_ref, m_ref):
    x = x_ref[...]
    w = w_ref[...]
    a = a_ref[...]
    h = jnp.dot(x, w, preferred_element_type=jnp.float32)
    h1 = h[:, :D]
    h2 = h[:, D:]
    ha_ref[0:N, :] = h1[:, :DH]
    ha_ref[N:, :] = h2[:, :DH]
    hb_ref[0:N, :] = h1[:, DH:]
    hb_ref[N:, :] = h2[:, DH:]
    s1 = jnp.sum(h1 * a[:, 0][None, :], axis=1)
    s2 = jnp.sum(h1 * a[:, 1][None, :], axis=1)
    s3 = jnp.sum(h2 * a[:, 2][None, :], axis=1)
    s4 = jnp.sum(h2 * a[:, 3][None, :], axis=1)
    s_ref[0, :] = s1
    s_ref[1, :] = s2
    s_ref[2, :] = s3
    s_ref[3, :] = s4
    m1 = jnp.max(s1) + jnp.max(s2)
    m1 = jnp.maximum(m1, 0.2 * m1)
    m2 = jnp.max(s3) + jnp.max(s4)
    m2 = jnp.maximum(m2, 0.2 * m2)
    m_ref[0, :] = jnp.full((128,), m1, dtype=jnp.float32)
    m_ref[1, :] = jnp.full((128,), m2, dtype=jnp.float32)


def _prologue(x, wcat, acat):
    return pl.pallas_call(
        _prologue_body,
        out_shape=[
            jax.ShapeDtypeStruct((2 * N, DH), jnp.float32),
            jax.ShapeDtypeStruct((2 * N, DH), jnp.float32),
            jax.ShapeDtypeStruct((4, N), jnp.float32),
            jax.ShapeDtypeStruct((2, 128), jnp.float32),
        ],
    )(x, wcat, acat)


def _epilogue_body(pa_ref, pb_ref, o_ref):
    o_ref[:, :DH] = pa_ref[0:N, :] + pa_ref[N:, :]
    o_ref[:, DH:] = pb_ref[0:N, :] + pb_ref[N:, :]


def _epilogue(pa, pb):
    return pl.pallas_call(
        _epilogue_body,
        out_shape=jax.ShapeDtypeStruct((N, D), jnp.float32),
    )(pa, pb)


def _sc_body(ha, hb, ecat, scat, mv, outa, outb,
             fbuf, rowbuf, stag, dstag,
             schunk, dchunk, gidx, cbuf, mvbuf, dn_sh, acc_sh, sem):
    cidx = lax.axis_index("c")
    wid = lax.axis_index("s")
    z16f = jnp.zeros((16,), jnp.float32)
    lane = lax.iota(jnp.int32, 16)

    s_base = pl.multiple_of(cidx * (2 * N), 2 * N)
    pltpu.sync_copy(scat.at[pl.ds(s_base, N)], fbuf.at[pl.ds(0, N)])
    pltpu.sync_copy(scat.at[pl.ds(s_base + N, N)], fbuf.at[pl.ds(N, N)])
    pltpu.sync_copy(mv, mvbuf)
    mvec = mvbuf[pl.ds(pl.multiple_of(cidx * 16, 16), 16)]

    def _z_stag(r, _):
        stag[r, :] = z16f
        dstag[r, :] = z16f
        return _
    lax.fori_loop(0, CHUNK, _z_stag, None)

    def _z_rowbuf(r, _):
        for jj in range(DH // 16):
            rowbuf[r, pl.ds(jj * 16, 16)] = z16f
        return _
    lax.fori_loop(0, 16, _z_rowbuf, None)

    nbase = wid * RPT
    nro = jnp.where(wid == NSUB - 1, 40, 39)

    def _zero_acc(i, _):
        roff = pl.multiple_of(nbase + i * 16, 8)
        pltpu.sync_copy(rowbuf.at[pl.ds(0, 16), :],
                        acc_sh.at[pl.ds(roff, 16), :])
        return _

    def _zero_dn(i, _):
        roff = pl.multiple_of(nbase + i * 16, 8)
        pltpu.sync_copy(stag.at[pl.ds(0, 16), :],
                        dn_sh.at[pl.ds(roff, 16), :])
        return _

    lax.fori_loop(0, nro, _zero_acc, None)
    lax.fori_loop(0, nro, _zero_dn, None)
    plsc.subcore_barrier()

    esrc_base = cidx * (2 * E) + wid * EPT
    edst_base = esrc_base + E

    def _edge_z(j):
        srcv = schunk[pl.ds(j * 16, 16)]
        dstv = dchunk[pl.ds(j * 16, 16)]
        ss = plsc.load_gather(fbuf, [srcv])
        sd = plsc.load_gather(fbuf, [dstv + N])
        v = ss + sd
        e = jnp.maximum(v, 0.2 * v)
        return srcv, dstv, e - mvec

    def _phase_a(i, _):
        eoff = pl.multiple_of(i * CHUNK, CHUNK)
        pltpu.sync_copy(ecat.at[pl.ds(esrc_base + eoff, CHUNK)], schunk)
        pltpu.sync_copy(ecat.at[pl.ds(edst_base + eoff, CHUNK)], dchunk)
        for j in range(VPC):
            zu1, zu2, z = _edge_z(j)
            q = jnp.exp(z * INV9)
            rows = lane + (j * 16)
            p = q
            for k in range(1, NSTEP + 1):
                plsc.store_scatter(stag, [rows, jnp.full((16,), k, jnp.int32)], p)
                if k < NSTEP:
                    p = p * q
        pltpu.sync_copy(stag, dn_sh.at[dchunk], add=True)
        return _

    lax.fori_loop(0, NCHUNK, _phase_a, None)
    plsc.subcore_barrier()

    hrow_base = cidx * N

    for half, (h_in, o_out) in enumerate(((ha, outa), (hb, outb))):
        def _phase_b(i, _):
            eoff = pl.multiple_of(i * CHUNK, CHUNK)
            pltpu.sync_copy(ecat.at[pl.ds(esrc_base + eoff, CHUNK)], schunk)
            pltpu.sync_copy(ecat.at[pl.ds(edst_base + eoff, CHUNK)], dchunk)
            for j in range(VPC):
                srcv = schunk[pl.ds(j * 16, 16)]
                gidx[pl.ds(j * 16, 16)] = srcv + hrow_base
            cp = pltpu.async_copy(h_in.at[gidx], rowbuf, sem)
            pltpu.sync_copy(dn_sh.at[dchunk], dstag)
            for j in range(VPC):
                zu1, zu2, z = _edge_z(j)
                q = jnp.exp(z * INV9)
                rows = lane + (j * 16)
                p = q
                c = jnp.zeros((16,), jnp.float32)
                for k in range(1, NSTEP + 1):
                    dn = plsc.load_gather(
                        dstag, [rows, jnp.full((16,), k, jnp.int32)])
                    c = c + (float(k) * INV9 * INV9) * p / (dn + 1e-16)
                    if k < NSTEP:
                        p = p * q
                cbuf[pl.ds(j * 16, 16)] = c
            cp.wait()

            def _scale(g, _):
                cv16 = cbuf[pl.ds(pl.multiple_of(g * 16, 16), 16)]
                for l in range(16):
                    cv = jnp.full((16,), cv16[l], jnp.float32)
                    r = g * 16 + l
                    for jj in range(DH // 16):
                        rowbuf[r, pl.ds(jj * 16, 16)] = (
                            rowbuf[r, pl.ds(jj * 16, 16)] * cv)
                return _
            lax.fori_loop(0, VPC, _scale, None)
            pltpu.sync_copy(rowbuf, acc_sh.at[dchunk], add=True)
            return _

        lax.fori_loop(0, NCHUNK, _phase_b, None)
        plsc.subcore_barrier()

        if half == 0:
            lax.fori_loop(0, 16, _z_rowbuf, None)

        def _readout(i, _):
            roff = pl.multiple_of(nbase + i * 16, 8)
            pltpu.sync_copy(acc_sh.at[pl.ds(roff, 16), :],
                            rowbuf.at[pl.ds(16, 16), :])
            pltpu.sync_copy(rowbuf.at[pl.ds(16, 16), :],
                            o_out.at[pl.ds(cidx * N + roff, 16), :])
            if half == 0:
                pltpu.sync_copy(rowbuf.at[pl.ds(0, 16), :],
                                acc_sh.at[pl.ds(roff, 16), :])
            return _
        lax.fori_loop(0, nro, _readout, None)
        if half == 0:
            plsc.subcore_barrier()


def _sc_call(ha, hb, ecat, scat, mv):
    mesh = plsc.VectorSubcoreMesh(
        core_axis_name="c", subcore_axis_name="s",
        num_cores=NCORES, num_subcores=NSUB)
    k = functools.partial(
        pl.kernel,
        out_type=[
            jax.ShapeDtypeStruct((2 * N, DH), jnp.float32),
            jax.ShapeDtypeStruct((2 * N, DH), jnp.float32),
        ],
        mesh=mesh,
        compiler_params=pltpu.CompilerParams(
            needs_layout_passes=False, use_tc_tiling_on_sc=False),
        scratch_types=[
            pltpu.VMEM((2 * N,), jnp.float32),
            pltpu.VMEM((CHUNK, DH), jnp.float32),
            pltpu.VMEM((CHUNK, 16), jnp.float32),
            pltpu.VMEM((CHUNK, 16), jnp.float32),
            pltpu.VMEM((CHUNK,), jnp.int32),
            pltpu.VMEM((CHUNK,), jnp.int32),
            pltpu.VMEM((CHUNK,), jnp.int32),
            pltpu.VMEM((CHUNK,), jnp.float32),
            pltpu.VMEM((32,), jnp.float32),
            pltpu.VMEM_SHARED((N, 16), jnp.float32),
            pltpu.VMEM_SHARED((N, DH), jnp.float32),
            pltpu.SemaphoreType.DMA,
        ],
    )(_sc_body)
    return k(ha, hb, ecat, scat, mv)


def kernel(x, edge_index_l1, edge_index_l2, W1, a_src1, a_dst1,
           W2, a_src2, a_dst2):
    wcat = jnp.concatenate([W1, W2], axis=1)
    acat = jnp.stack([a_src1, a_dst1, a_src2, a_dst2], axis=1)
    ha, hb, s4, m2 = _prologue(x, wcat, acat)
    scat = s4.reshape(4 * N)
    mv = jnp.concatenate([m2[0, :16], m2[1, :16]])
    ecat = jnp.concatenate(
        [edge_index_l1.reshape(-1), edge_index_l2.reshape(-1)])
    pa, pb = _sc_call(ha, hb, ecat, scat, mv)
    return _epilogue(pa, pb)


# speedup vs baseline: 173.1107x; 1.4829x over previous
import functools

import jax
import jax.numpy as jnp
from jax import lax
from jax.experimental import pallas as pl
from jax.experimental.pallas import tpu as pltpu
from jax.experimental.pallas import tpu_sc as plsc

N = 10000
D = 128
E = 320000
NSTEP = 9
INV9 = 1.0 / 9.0

NCORES = 2
NSUB = 16
EPT = E // NSUB
RPT = 624
CHUNK = 80
NCHUNK = EPT // CHUNK
VPC = CHUNK // 16


def _prologue_body(x_ref, w_ref, a_ref, h_ref, s_ref, m_ref):
    x = x_ref[...]
    w = w_ref[...]
    a = a_ref[...]
    h = jnp.dot(x, w, preferred_element_type=jnp.float32)
    h1 = h[:, :D]
    h2 = h[:, D:]
    h_ref[0:N, :] = h1
    h_ref[N:, :] = h2
    s1 = jnp.sum(h1 * a[:, 0][None, :], axis=1)
    s2 = jnp.sum(h1 * a[:, 1][None, :], axis=1)
    s3 = jnp.sum(h2 * a[:, 2][None, :], axis=1)
    s4 = jnp.sum(h2 * a[:, 3][None, :], axis=1)
    s_ref[0, :] = s1
    s_ref[1, :] = s2
    s_ref[2, :] = s3
    s_ref[3, :] = s4
    m1 = jnp.max(s1) + jnp.max(s2)
    m1 = jnp.maximum(m1, 0.2 * m1)
    m2 = jnp.max(s3) + jnp.max(s4)
    m2 = jnp.maximum(m2, 0.2 * m2)
    m_ref[0, :] = jnp.full((128,), m1, dtype=jnp.float32)
    m_ref[1, :] = jnp.full((128,), m2, dtype=jnp.float32)


def _prologue(x, wcat, acat):
    return pl.pallas_call(
        _prologue_body,
        out_shape=[
            jax.ShapeDtypeStruct((2 * N, D), jnp.float32),
            jax.ShapeDtypeStruct((4, N), jnp.float32),
            jax.ShapeDtypeStruct((2, 128), jnp.float32),
        ],
    )(x, wcat, acat)


def _epilogue_body(p_ref, o_ref):
    o_ref[...] = p_ref[0:N, :] + p_ref[N:, :]


def _epilogue(p):
    return pl.pallas_call(
        _epilogue_body,
        out_shape=jax.ShapeDtypeStruct((N, D), jnp.float32),
    )(p)


def _sc_body(h2n, ecat, scat, mv, out,
             fbuf, rowbuf, stag, dstag,
             schunk, dchunk, gidx, cbuf, mvbuf, dn_sh, acc_sh, sem):
    cidx = lax.axis_index("c")
    wid = lax.axis_index("s")
    z16f = jnp.zeros((16,), jnp.float32)
    lane = lax.iota(jnp.int32, 16)

    s_base = pl.multiple_of(cidx * (2 * N), 2 * N)
    pltpu.sync_copy(scat.at[pl.ds(s_base, N)], fbuf.at[pl.ds(0, N)])
    pltpu.sync_copy(scat.at[pl.ds(s_base + N, N)], fbuf.at[pl.ds(N, N)])
    pltpu.sync_copy(mv, mvbuf)
    mvec = mvbuf[pl.ds(pl.multiple_of(cidx * 16, 16), 16)]

    def _z_stag(r, _):
        stag[r, :] = z16f
        dstag[r, :] = z16f
        return _
    lax.fori_loop(0, CHUNK, _z_stag, None)

    def _z_rowbuf(r, _):
        for jj in range(D // 16):
            rowbuf[r, pl.ds(jj * 16, 16)] = z16f
        return _
    lax.fori_loop(0, 16, _z_rowbuf, None)

    nbase = wid * RPT
    nro = jnp.where(wid == NSUB - 1, 40, 39)

    def _zero_shared(i, _):
        roff = pl.multiple_of(nbase + i * 16, 8)
        pltpu.sync_copy(rowbuf.at[pl.ds(0, 16), :],
                        acc_sh.at[pl.ds(roff, 16), :])
        pltpu.sync_copy(stag.at[pl.ds(0, 16), :],
                        dn_sh.at[pl.ds(roff, 16), :])
        return _

    lax.fori_loop(0, nro, _zero_shared, None)
    plsc.subcore_barrier()

    esrc_base = cidx * (2 * E) + wid * EPT
    edst_base = esrc_base + E

    def _edge_z(j):
        srcv = schunk[pl.ds(j * 16, 16)]
        dstv = dchunk[pl.ds(j * 16, 16)]
        ss = plsc.load_gather(fbuf, [srcv])
        sd = plsc.load_gather(fbuf, [dstv + N])
        v = ss + sd
        e = jnp.maximum(v, 0.2 * v)
        return e - mvec

    def _phase_a(i, _):
        eoff = pl.multiple_of(i * CHUNK, CHUNK)
        pltpu.sync_copy(ecat.at[pl.ds(esrc_base + eoff, CHUNK)], schunk)
        pltpu.sync_copy(ecat.at[pl.ds(edst_base + eoff, CHUNK)], dchunk)
        for j in range(VPC):
            z = _edge_z(j)
            q = jnp.exp(z * INV9)
            rows = lane + (j * 16)
            p = q
            for k in range(1, NSTEP + 1):
                plsc.store_scatter(stag, [rows, jnp.full((16,), k, jnp.int32)], p)
                if k < NSTEP:
                    p = p * q
        pltpu.sync_copy(stag, dn_sh.at[dchunk], add=True)
        return _

    lax.fori_loop(0, NCHUNK, _phase_a, None)
    plsc.subcore_barrier()

    hrow_base = cidx * N

    def _phase_b(i, _):
        eoff = pl.multiple_of(i * CHUNK, CHUNK)
        pltpu.sync_copy(ecat.at[pl.ds(esrc_base + eoff, CHUNK)], schunk)
        pltpu.sync_copy(ecat.at[pl.ds(edst_base + eoff, CHUNK)], dchunk)
        for j in range(VPC):
            srcv = schunk[pl.ds(j * 16, 16)]
            gidx[pl.ds(j * 16, 16)] = srcv + hrow_base
        cp = pltpu.async_copy(h2n.at[gidx], rowbuf, sem)
        pltpu.sync_copy(dn_sh.at[dchunk], dstag)
        for j in range(VPC):
            z = _edge_z(j)
            q = jnp.exp(z * INV9)
            rows = lane + (j * 16)
            p = q
            c = jnp.zeros((16,), jnp.float32)
            for k in range(1, NSTEP + 1):
                dn = plsc.load_gather(
                    dstag, [rows, jnp.full((16,), k, jnp.int32)])
                c = c + (float(k) * INV9 * INV9) * p / (dn + 1e-16)
                if k < NSTEP:
                    p = p * q
            cbuf[pl.ds(j * 16, 16)] = c
        cp.wait()

        def _scale(g, _):
            cv16 = cbuf[pl.ds(pl.multiple_of(g * 16, 16), 16)]
            for l in range(16):
                cv = jnp.full((16,), cv16[l], jnp.float32)
                r = g * 16 + l
                for jj in range(D // 16):
                    rowbuf[r, pl.ds(jj * 16, 16)] = (
                        rowbuf[r, pl.ds(jj * 16, 16)] * cv)
            return _
        lax.fori_loop(0, VPC, _scale, None)
        pltpu.sync_copy(rowbuf, acc_sh.at[dchunk], add=True)
        return _

    lax.fori_loop(0, NCHUNK, _phase_b, None)
    plsc.subcore_barrier()

    def _readout(i, _):
        roff = pl.multiple_of(nbase + i * 16, 8)
        pltpu.sync_copy(acc_sh.at[pl.ds(roff, 16), :],
                        rowbuf.at[pl.ds(16, 16), :])
        pltpu.sync_copy(rowbuf.at[pl.ds(16, 16), :],
                        out.at[pl.ds(cidx * N + roff, 16), :])
        return _
    lax.fori_loop(0, nro, _readout, None)


def _sc_call(h2n, ecat, scat, mv):
    mesh = plsc.VectorSubcoreMesh(
        core_axis_name="c", subcore_axis_name="s",
        num_cores=NCORES, num_subcores=NSUB)
    k = functools.partial(
        pl.kernel,
        out_type=jax.ShapeDtypeStruct((2 * N, D), jnp.float32),
        mesh=mesh,
        compiler_params=pltpu.CompilerParams(
            needs_layout_passes=False, use_tc_tiling_on_sc=False,
            internal_scratch_in_bytes=128 * 1024),
        scratch_types=[
            pltpu.VMEM((2 * N,), jnp.float32),
            pltpu.VMEM((CHUNK, D), jnp.float32),
            pltpu.VMEM((CHUNK, 16), jnp.float32),
            pltpu.VMEM((CHUNK, 16), jnp.float32),
            pltpu.VMEM((CHUNK,), jnp.int32),
            pltpu.VMEM((CHUNK,), jnp.int32),
            pltpu.VMEM((CHUNK,), jnp.int32),
            pltpu.VMEM((CHUNK,), jnp.float32),
            pltpu.VMEM((32,), jnp.float32),
            pltpu.VMEM_SHARED((N, 16), jnp.float32),
            pltpu.VMEM_SHARED((N, D), jnp.float32),
            pltpu.SemaphoreType.DMA,
        ],
    )(_sc_body)
    return k(h2n, ecat, scat, mv)


def kernel(x, edge_index_l1, edge_index_l2, W1, a_src1, a_dst1,
           W2, a_src2, a_dst2):
    wcat = jnp.concatenate([W1, W2], axis=1)
    acat = jnp.stack([a_src1, a_dst1, a_src2, a_dst2], axis=1)
    h2n, s4, m2 = _prologue(x, wcat, acat)
    scat = s4.reshape(4 * N)
    mv = jnp.concatenate([m2[0, :16], m2[1, :16]])
    ecat = jnp.concatenate(
        [edge_index_l1.reshape(-1), edge_index_l2.reshape(-1)])
    partials = _sc_call(h2n, ecat, scat, mv)
    return _epilogue(partials)


# speedup vs baseline: 373.0432x; 2.1549x over previous
import functools

import jax
import jax.numpy as jnp
from jax import lax
from jax.experimental import pallas as pl
from jax.experimental.pallas import tpu as pltpu
from jax.experimental.pallas import tpu_sc as plsc

N = 10000
D = 128
E = 320000
NSTEP = 9
INV9 = 1.0 / 9.0

NCORES = 2
NSUB = 16
EPT = E // NSUB
RPT = 624
CHUNK = 80
NCHUNK = EPT // CHUNK
NPAIR = NCHUNK // 2
VPC = CHUNK // 16
RZ = 48
NPAD = CHUNK * D


def _prologue_body(x_ref, w_ref, a_ref, h_ref, s_ref, m_ref):
    x = x_ref[...]
    w = w_ref[...]
    a = a_ref[...]
    h = jnp.dot(x, w, preferred_element_type=jnp.float32)
    h1 = h[:, :D]
    h2 = h[:, D:]
    h_ref[0:N, :] = h1
    h_ref[N:, :] = h2
    s1 = jnp.sum(h1 * a[:, 0][None, :], axis=1)
    s2 = jnp.sum(h1 * a[:, 1][None, :], axis=1)
    s3 = jnp.sum(h2 * a[:, 2][None, :], axis=1)
    s4 = jnp.sum(h2 * a[:, 3][None, :], axis=1)
    s_ref[0, :] = s1
    s_ref[1, :] = s2
    s_ref[2, :] = s3
    s_ref[3, :] = s4
    m1 = jnp.max(s1) + jnp.max(s2)
    m1 = jnp.maximum(m1, 0.2 * m1)
    m2 = jnp.max(s3) + jnp.max(s4)
    m2 = jnp.maximum(m2, 0.2 * m2)
    m_ref[0, :] = jnp.full((128,), m1, dtype=jnp.float32)
    m_ref[1, :] = jnp.full((128,), m2, dtype=jnp.float32)


def _prologue(x, wcat, acat):
    return pl.pallas_call(
        _prologue_body,
        out_shape=[
            jax.ShapeDtypeStruct((2 * N, D), jnp.float32),
            jax.ShapeDtypeStruct((4, N), jnp.float32),
            jax.ShapeDtypeStruct((2, 128), jnp.float32),
        ],
    )(x, wcat, acat)


def _epilogue_body(p_ref, o_ref):
    o_ref[...] = p_ref[0:N, :] + p_ref[N:, :]


def _epilogue(p):
    return pl.pallas_call(
        _epilogue_body,
        out_shape=jax.ShapeDtypeStruct((N, D), jnp.float32),
    )(p)


def _sc_body(h2n, ecat, scat4, mv, out, zout,
             rowbuf, stag, dstag,
             schunk, dchunk, gidx, sidx, zbuf, cbuf, mvbuf, dn_sh, acc_sh,
             sem_e0, sem_e1, sem_h0, sem_h1, sem_d0, sem_d1, sem_s0, sem_s1,
             sem_z0, sem_z1):
    cidx = lax.axis_index("c")
    wid = lax.axis_index("s")
    z16f = jnp.zeros((16,), jnp.float32)
    lane = lax.iota(jnp.int32, 16)
    sem_e = (sem_e0, sem_e1)
    sem_h = (sem_h0, sem_h1)
    sem_d = (sem_d0, sem_d1)
    sem_s = (sem_s0, sem_s1)
    sem_z = (sem_z0, sem_z1)

    pltpu.sync_copy(scat4.at[cidx * 2], rowbuf.at[0])
    pltpu.sync_copy(scat4.at[cidx * 2 + 1], rowbuf.at[1])
    pltpu.sync_copy(mv, mvbuf)
    mvec = mvbuf[pl.ds(pl.multiple_of(cidx * 16, 16), 16)]

    def _z_stag(r, _):
        stag[0, r, :] = z16f
        stag[1, r, :] = z16f
        return _
    lax.fori_loop(0, CHUNK, _z_stag, None)

    nbase = wid * RPT

    def _zero_dn(i, _):
        roff = pl.multiple_of(nbase + i * RZ, 8)
        pltpu.sync_copy(stag.at[0, pl.ds(0, RZ), :],
                        dn_sh.at[pl.ds(roff, RZ), :])
        return _

    lax.fori_loop(0, RPT // RZ, _zero_dn, None)

    @pl.when(wid == NSUB - 1)
    def _zero_dn_tail():
        roff = pl.multiple_of(nbase + RPT, 8)
        pltpu.sync_copy(stag.at[0, pl.ds(0, 16), :],
                        dn_sh.at[pl.ds(roff, 16), :])

    plsc.subcore_barrier()

    esrc_base = cidx * (2 * E) + wid * EPT
    edst_base = esrc_base + E
    zedge_base = cidx * E + wid * EPT
    hrow_base = cidx * N

    def _eoff(i):
        return pl.multiple_of(i * CHUNK, CHUNK)

    def _load_edges(i, sl):
        pltpu.async_copy(ecat.at[pl.ds(esrc_base + _eoff(i), CHUNK)],
                         schunk.at[sl], sem_e[sl])
        pltpu.async_copy(ecat.at[pl.ds(edst_base + _eoff(i), CHUNK)],
                         dchunk.at[sl], sem_e[sl])

    def _wait_edges(i, sl):
        pltpu.make_async_copy(ecat.at[pl.ds(esrc_base + _eoff(i), CHUNK)],
                              schunk.at[sl], sem_e[sl]).wait()
        pltpu.make_async_copy(ecat.at[pl.ds(edst_base + _eoff(i), CHUNK)],
                              dchunk.at[sl], sem_e[sl]).wait()

    def _a_compute(sl):
        for j in range(VPC):
            srcv = schunk[sl, pl.ds(j * 16, 16)]
            dstv = dchunk[sl, pl.ds(j * 16, 16)]
            ss = plsc.load_gather(
                rowbuf, [jnp.zeros((16,), jnp.int32),
                         lax.shift_right_logical(srcv, 7),
                         lax.bitwise_and(srcv, 127)])
            sd = plsc.load_gather(
                rowbuf, [jnp.ones((16,), jnp.int32),
                         lax.shift_right_logical(dstv, 7),
                         lax.bitwise_and(dstv, 127)])
            v = ss + sd
            e = jnp.maximum(v, 0.2 * v)
            z = e - mvec
            zbuf[sl, pl.ds(j * 16, 16)] = z
            q = jnp.exp(z * INV9)
            rows = lane + (j * 16)
            p = q
            for k in range(1, NSTEP + 1):
                plsc.store_scatter(
                    stag.at[sl], [rows, jnp.full((16,), k, jnp.int32)], p)
                if k < NSTEP:
                    p = p * q
            sidx[sl, pl.ds(j * 16, 16)] = dstv

    def _a_out(i, sl):
        pltpu.async_copy(stag.at[sl], dn_sh.at[sidx.at[sl]], sem_s[sl],
                         add=True)
        pltpu.async_copy(zbuf.at[sl],
                         zout.at[pl.ds(zedge_base + _eoff(i), CHUNK)],
                         sem_z[sl])

    def _a_out_wait(i, sl):
        pltpu.make_async_copy(stag.at[sl], dn_sh.at[sidx.at[sl]],
                              sem_s[sl]).wait()
        pltpu.make_async_copy(zbuf.at[sl],
                              zout.at[pl.ds(zedge_base + _eoff(i), CHUNK)],
                              sem_z[sl]).wait()

    _load_edges(0, 0)

    def _phase_a(t, _):
        a = 2 * t
        _load_edges(a + 1, 1)
        _wait_edges(a, 0)

        @pl.when(t > 0)
        def _():
            _a_out_wait(a - 2, 0)
        _a_compute(0)
        _a_out(a, 0)

        @pl.when(t < NPAIR - 1)
        def _():
            _load_edges(a + 2, 0)
        _wait_edges(a + 1, 1)

        @pl.when(t > 0)
        def _():
            _a_out_wait(a - 1, 1)
        _a_compute(1)
        _a_out(a + 1, 1)
        return _

    lax.fori_loop(0, NPAIR, _phase_a, None)
    _a_out_wait(NCHUNK - 2, 0)
    _a_out_wait(NCHUNK - 1, 1)
    plsc.subcore_barrier()

    def _z_rowbuf(r, _):
        for jj in range(D // 16):
            rowbuf[0, r, pl.ds(jj * 16, 16)] = z16f
        return _
    lax.fori_loop(0, RZ, _z_rowbuf, None)

    def _zero_acc(i, _):
        roff = pl.multiple_of(nbase + i * RZ, 8)
        pltpu.sync_copy(rowbuf.at[0, pl.ds(0, RZ), :],
                        acc_sh.at[pl.ds(roff, RZ), :])
        return _

    lax.fori_loop(0, RPT // RZ, _zero_acc, None)

    @pl.when(wid == NSUB - 1)
    def _zero_acc_tail():
        roff = pl.multiple_of(nbase + RPT, 8)
        pltpu.sync_copy(rowbuf.at[0, pl.ds(0, 16), :],
                        acc_sh.at[pl.ds(roff, 16), :])

    plsc.subcore_barrier()

    def _load_z(i, sl):
        pltpu.async_copy(zout.at[pl.ds(zedge_base + _eoff(i), CHUNK)],
                         zbuf.at[sl], sem_z[sl])

    def _wait_z(i, sl):
        pltpu.make_async_copy(zout.at[pl.ds(zedge_base + _eoff(i), CHUNK)],
                              zbuf.at[sl], sem_z[sl]).wait()

    def _b_gath(i, sl):
        _wait_edges(i, sl)
        for j in range(VPC):
            srcv = schunk[sl, pl.ds(j * 16, 16)]
            gidx[sl, pl.ds(j * 16, 16)] = srcv + hrow_base
            sidx[sl, pl.ds(j * 16, 16)] = dchunk[sl, pl.ds(j * 16, 16)]
        pltpu.async_copy(h2n.at[gidx.at[sl]], rowbuf.at[sl], sem_h[sl])
        pltpu.async_copy(dn_sh.at[dchunk.at[sl]], dstag.at[sl], sem_d[sl])

    def _b_comp(i, sl):
        pltpu.make_async_copy(dn_sh.at[dchunk.at[sl]], dstag.at[sl],
                              sem_d[sl]).wait()
        _wait_z(i, sl)
        for j in range(VPC):
            z = zbuf[sl, pl.ds(j * 16, 16)]
            q = jnp.exp(z * INV9)
            rows = lane + (j * 16)
            p = q
            c = jnp.zeros((16,), jnp.float32)
            for k in range(1, NSTEP + 1):
                dn = plsc.load_gather(
                    dstag.at[sl], [rows, jnp.full((16,), k, jnp.int32)])
                c = c + (float(k) * INV9 * INV9) * p / (dn + 1e-16)
                if k < NSTEP:
                    p = p * q
            cbuf[pl.ds(j * 16, 16)] = c
        pltpu.make_async_copy(h2n.at[gidx.at[sl]], rowbuf.at[sl],
                              sem_h[sl]).wait()

        def _scale(g, _):
            cv16 = cbuf[pl.ds(pl.multiple_of(g * 16, 16), 16)]
            for l in range(16):
                cv = jnp.full((16,), cv16[l], jnp.float32)
                r = g * 16 + l
                for jj in range(D // 16):
                    rowbuf[sl, r, pl.ds(jj * 16, 16)] = (
                        rowbuf[sl, r, pl.ds(jj * 16, 16)] * cv)
            return _
        lax.fori_loop(0, VPC, _scale, None)
        pltpu.async_copy(rowbuf.at[sl], acc_sh.at[sidx.at[sl]], sem_s[sl],
                         add=True)

    def _b_scat_wait(sl):
        pltpu.make_async_copy(rowbuf.at[sl], acc_sh.at[sidx.at[sl]],
                              sem_s[sl]).wait()

    _load_edges(0, 0)
    _load_z(0, 0)
    _b_gath(0, 0)

    def _phase_b(t, _):
        a = 2 * t

        @pl.when(t > 0)
        def _():
            _b_scat_wait(1)
        _load_edges(a + 1, 1)
        _load_z(a + 1, 1)
        _b_gath(a + 1, 1)
        _b_comp(a, 0)
        _b_comp(a + 1, 1)

        @pl.when(t < NPAIR - 1)
        def _():
            _b_scat_wait(0)
            _load_edges(a + 2, 0)
            _load_z(a + 2, 0)
            _b_gath(a + 2, 0)
        return _

    lax.fori_loop(0, NPAIR, _phase_b, None)
    _b_scat_wait(0)
    _b_scat_wait(1)
    plsc.subcore_barrier()

    def _readout(i, _):
        roff = pl.multiple_of(nbase + i * RZ, 8)
        pltpu.sync_copy(acc_sh.at[pl.ds(roff, RZ), :],
                        rowbuf.at[0, pl.ds(0, RZ), :])
        pltpu.sync_copy(rowbuf.at[0, pl.ds(0, RZ), :],
                        out.at[pl.ds(cidx * N + roff, RZ), :])
        return _
    lax.fori_loop(0, RPT // RZ, _readout, None)

    @pl.when(wid == NSUB - 1)
    def _readout_tail():
        roff = pl.multiple_of(nbase + RPT, 8)
        pltpu.sync_copy(acc_sh.at[pl.ds(roff, 16), :],
                        rowbuf.at[0, pl.ds(0, 16), :])
        pltpu.sync_copy(rowbuf.at[0, pl.ds(0, 16), :],
                        out.at[pl.ds(cidx * N + roff, 16), :])


def _sc_call(h2n, ecat, scat4, mv):
    mesh = plsc.VectorSubcoreMesh(
        core_axis_name="c", subcore_axis_name="s",
        num_cores=NCORES, num_subcores=NSUB)
    k = functools.partial(
        pl.kernel,
        out_type=[
            jax.ShapeDtypeStruct((2 * N, D), jnp.float32),
            jax.ShapeDtypeStruct((2 * E,), jnp.float32),
        ],
        mesh=mesh,
        compiler_params=pltpu.CompilerParams(
            needs_layout_passes=False, use_tc_tiling_on_sc=False,
            internal_scratch_in_bytes=128 * 1024),
        scratch_types=[
            pltpu.VMEM((2, CHUNK, D), jnp.float32),
            pltpu.VMEM((2, CHUNK, 16), jnp.float32),
            pltpu.VMEM((2, CHUNK, 16), jnp.float32),
            pltpu.VMEM((2, CHUNK), jnp.int32),
            pltpu.VMEM((2, CHUNK), jnp.int32),
            pltpu.VMEM((2, CHUNK), jnp.int32),
            pltpu.VMEM((2, CHUNK), jnp.int32),
            pltpu.VMEM((2, CHUNK), jnp.float32),
            pltpu.VMEM((CHUNK,), jnp.float32),
            pltpu.VMEM((32,), jnp.float32),
            pltpu.VMEM_SHARED((N, 16), jnp.float32),
            pltpu.VMEM_SHARED((N, D), jnp.float32),
            pltpu.SemaphoreType.DMA,
            pltpu.SemaphoreType.DMA,
            pltpu.SemaphoreType.DMA,
            pltpu.SemaphoreType.DMA,
            pltpu.SemaphoreType.DMA,
            pltpu.SemaphoreType.DMA,
            pltpu.SemaphoreType.DMA,
            pltpu.SemaphoreType.DMA,
            pltpu.SemaphoreType.DMA,
            pltpu.SemaphoreType.DMA,
        ],
    )(_sc_body)
    return k(h2n, ecat, scat4, mv)


def kernel(x, edge_index_l1, edge_index_l2, W1, a_src1, a_dst1,
           W2, a_src2, a_dst2):
    wcat = jnp.concatenate([W1, W2], axis=1)
    acat = jnp.stack([a_src1, a_dst1, a_src2, a_dst2], axis=1)
    h2n, s4, m2 = _prologue(x, wcat, acat)
    scat4 = jnp.pad(s4, ((0, 0), (0, NPAD - N))).reshape(4, CHUNK, D)
    mv = jnp.concatenate([m2[0, :16], m2[1, :16]])
    ecat = jnp.concatenate(
        [edge_index_l1.reshape(-1), edge_index_l2.reshape(-1)])
    partials, _zscratch = _sc_call(h2n, ecat, scat4, mv)
    return _epilogue(partials)
